# Initial kernel scaffold; baseline (speedup 1.0000x reference)
import jax
import jax.numpy as jnp
from jax.experimental import pallas as pl


def kernel(x, pe):
    raise NotImplementedError("write your pallas kernel here")


# speedup vs baseline: 1.5204x; 1.5204x over previous
import functools

import jax
import jax.numpy as jnp
from jax import lax
from jax.experimental import pallas as pl
from jax.experimental.pallas import tpu as pltpu
from jax.experimental.pallas import tpu_sc as plsc

N_DAYS = 366
D_MODEL = 256
BATCH = 16384

NC = 2
NS = 16
NW = NC * NS
B_PER_W = BATCH // NW
CHUNK = 128
N_CHUNK = B_PER_W // CHUNK
IDX_ROWS = BATCH // CHUNK

_mesh = plsc.VectorSubcoreMesh(core_axis_name="c", subcore_axis_name="s")


@functools.partial(
    pl.kernel,
    mesh=_mesh,
    out_type=jax.ShapeDtypeStruct((BATCH, D_MODEL), jnp.float32),
    scratch_types=[
        pltpu.VMEM((N_CHUNK, CHUNK), jnp.int32),
        pltpu.VMEM((CHUNK, D_MODEL), jnp.float32),
        pltpu.VMEM((CHUNK, D_MODEL), jnp.float32),
        pltpu.SemaphoreType.DMA,
        pltpu.SemaphoreType.DMA,
    ],
)
def _gather_kernel(x_hbm, pe_hbm, out_hbm, idx_v, rows0, rows1, sem0, sem1):
    wid = lax.axis_index("s") * NC + lax.axis_index("c")
    row0 = wid * N_CHUNK
    base = wid * B_PER_W

    pltpu.sync_copy(x_hbm.at[pl.ds(row0, N_CHUNK)], idx_v)

    for j in range(N_CHUNK):
        for k in range(CHUNK // 16):
            v = idx_v[j, pl.ds(k * 16, 16)]
            idx_v[j, pl.ds(k * 16, 16)] = jnp.where(v == 0, N_DAYS - 1, v - 1)

    bufs = (rows0, rows1)
    sems = (sem0, sem1)
    copies = [None] * N_CHUNK
    copies[0] = pltpu.async_copy(pe_hbm.at[idx_v.at[0]], bufs[0], sems[0])
    for j in range(N_CHUNK):
        if j + 1 < N_CHUNK:
            copies[j + 1] = pltpu.async_copy(
                pe_hbm.at[idx_v.at[j + 1]], bufs[(j + 1) % 2], sems[(j + 1) % 2]
            )
        copies[j].wait()
        pltpu.sync_copy(bufs[j % 2], out_hbm.at[pl.ds(base + j * CHUNK, CHUNK)])


def kernel(x, pe):
    x32 = x.astype(jnp.int32).reshape(IDX_ROWS, CHUNK)
    return _gather_kernel(x32, pe)


# speedup vs baseline: 1.5630x; 1.0280x over previous
import functools

import jax
import jax.numpy as jnp
from jax import lax
from jax.experimental import pallas as pl
from jax.experimental.pallas import tpu as pltpu
from jax.experimental.pallas import tpu_sc as plsc

N_DAYS = 366
D_MODEL = 256
BATCH = 16384

NC = 2
NS = 16
NW = NC * NS
B_PER_W = BATCH // NW
CHUNK = 128
N_CHUNK = B_PER_W // CHUNK
IDX_ROWS = BATCH // CHUNK

_mesh = plsc.VectorSubcoreMesh(core_axis_name="c", subcore_axis_name="s")


NBUF = 3


@functools.partial(
    pl.kernel,
    mesh=_mesh,
    out_type=jax.ShapeDtypeStruct((BATCH, D_MODEL), jnp.float32),
    scratch_types=[
        pltpu.VMEM((N_CHUNK, CHUNK), jnp.int32),
        *[pltpu.VMEM((CHUNK, D_MODEL), jnp.float32) for _ in range(NBUF)],
        *[pltpu.SemaphoreType.DMA for _ in range(2 * NBUF)],
    ],
)
def _gather_kernel(x_hbm, pe_hbm, out_hbm, idx_v, *scratch):
    bufs = scratch[:NBUF]
    gsems = scratch[NBUF:2 * NBUF]
    ssems = scratch[2 * NBUF:]
    wid = lax.axis_index("s") * NC + lax.axis_index("c")
    row0 = wid * N_CHUNK
    base = wid * B_PER_W

    pltpu.sync_copy(x_hbm.at[pl.ds(row0, N_CHUNK)], idx_v)

    for j in range(N_CHUNK):
        for k in range(CHUNK // 16):
            v = idx_v[j, pl.ds(k * 16, 16)]
            idx_v[j, pl.ds(k * 16, 16)] = jnp.where(v == 0, N_DAYS - 1, v - 1)

    def gather(j):
        return pltpu.async_copy(pe_hbm.at[idx_v.at[j]], bufs[j % NBUF], gsems[j % NBUF])

    def scatter(j):
        return pltpu.async_copy(
            bufs[j % NBUF], out_hbm.at[pl.ds(base + j * CHUNK, CHUNK)], ssems[j % NBUF]
        )

    gcp = [None] * N_CHUNK
    scp = [None] * N_CHUNK
    for j in range(min(NBUF, N_CHUNK)):
        gcp[j] = gather(j)
    for j in range(N_CHUNK):
        gcp[j].wait()
        scp[j] = scatter(j)
        if j + NBUF < N_CHUNK:
            scp[j].wait()
            gcp[j + NBUF] = gather(j + NBUF)
    for j in range(max(0, N_CHUNK - NBUF), N_CHUNK):
        scp[j].wait()


def kernel(x, pe):
    x32 = x.astype(jnp.int32).reshape(IDX_ROWS, CHUNK)
    return _gather_kernel(x32, pe)
